# baseline (speedup 1.0000x reference)
import functools

import jax
import jax.numpy as jnp
from jax import lax
from jax.experimental import pallas as pl
from jax.experimental.pallas import tpu as pltpu
from jax.experimental.pallas import tpu_sc as plsc

CROP_RATIO = 0.8


@functools.lru_cache(maxsize=None)
def _crop_call(B, L, C, crop_len):
    info = plsc.get_sparse_core_info()
    NC, NS, NL = info.num_cores, info.num_subcores, info.num_lanes
    NW = NC * NS
    assert B == NW, "one subcore per batch element"
    assert B % NL == 0

    max_rows = (131071 - B - 1024) // (2 * C)
    nch = -(-crop_len // max_rows)
    chunk = -(-crop_len // nch)
    sizes = []
    off = 0
    offs = []
    while off < crop_len:
        sz = min(chunk, crop_len - off)
        offs.append(off)
        sizes.append(sz)
        off += sz
    nch = len(sizes)

    mesh = plsc.VectorSubcoreMesh(core_axis_name="c", subcore_axis_name="s")

    @functools.partial(
        pl.kernel,
        mesh=mesh,
        compiler_params=pltpu.CompilerParams(
            use_tc_tiling_on_sc=False, needs_layout_passes=False
        ),
        out_type=jax.ShapeDtypeStruct((B, crop_len, C), jnp.float32),
        scratch_types=[
            pltpu.VMEM((B,), jnp.int32),
            pltpu.VMEM((chunk, C), jnp.float32),
            pltpu.VMEM((chunk, C), jnp.float32),
            pltpu.SemaphoreType.DMA,
            pltpu.SemaphoreType.DMA,
            pltpu.SemaphoreType.DMA,
            pltpu.SemaphoreType.DMA,
        ],
    )
    def k(x_hbm, start_hbm, out_hbm, start_v, buf0, buf1, rs0, rs1, ws0, ws1):
        wid = lax.axis_index("c") * NS + lax.axis_index("s")
        pltpu.sync_copy(start_hbm, start_v)

        lane = lax.iota(jnp.int32, NL)
        group = jnp.where(wid < NL, start_v[pl.ds(0, NL)], start_v[pl.ds(NL, NL)])
        s = jnp.max(jnp.where(lane == wid % NL, group, 0))

        bufs = (buf0, buf1)
        rsems = (rs0, rs1)
        wsems = (ws0, ws1)

        def rd(i):
            return pltpu.make_async_copy(
                x_hbm.at[wid, pl.ds(s + offs[i], sizes[i]), :],
                bufs[i % 2].at[pl.ds(0, sizes[i]), :],
                rsems[i % 2],
            )

        def wr(i):
            return pltpu.make_async_copy(
                bufs[i % 2].at[pl.ds(0, sizes[i]), :],
                out_hbm.at[wid, pl.ds(offs[i], sizes[i]), :],
                wsems[i % 2],
            )

        rd(0).start()
        for i in range(nch):
            rd(i).wait()
            if i + 1 < nch:
                if i >= 1:
                    wr(i - 1).wait()
                rd(i + 1).start()
            wr(i).start()
        if nch >= 2:
            wr(nch - 2).wait()
        wr(nch - 1).wait()

    return k


def kernel(x):
    B, L, C = x.shape
    crop_len = int(L * CROP_RATIO)
    start = jax.random.randint(
        jax.random.key(1), (B,), 0, L - crop_len + 1
    ).astype(jnp.int32)
    return _crop_call(B, L, C, crop_len)(x, start)
